# Initial kernel scaffold; baseline (speedup 1.0000x reference)
import jax
import jax.numpy as jnp
from jax.experimental import pallas as pl


def kernel(x, edge_index, stage1_nodes, stage2_nodes, stage3_nodes, Wl1, bl1, Wr1, Wl2, bl2, Wr2, Wl3, bl3, Wr3, fc1_W, fc1_b, ln_g, ln_b, prelu_a, fc2_W, fc2_b):
    raise NotImplementedError("write your pallas kernel here")


# speedup vs baseline: 96.2541x; 96.2541x over previous
import functools

import jax
import jax.numpy as jnp
from jax import lax
from jax.experimental import pallas as pl
from jax.experimental.pallas import tpu as pltpu
from jax.experimental.pallas import tpu_sc as plsc

N = 100000
E = 1600000
D = 32
NSLOT = 48
SLOT64 = 64
NTGT = 24
BITW = 3136
BIG = 2 ** 30

TARGET_IDS = (65, 66, 67, 73, 77, 78, 79, 80, 81, 82, 83,
              90, 91, 92, 93, 94, 95, 96, 97, 98, 99, 100, 101)

_info = plsc.get_sparse_core_info()
NC, NS = _info.num_cores, _info.num_subcores
NW = NC * NS
EPW = E // NW
NCHUNK = EPW // 16


def _binsearch64(slots_v, keys):
    pos = jnp.zeros((16,), jnp.int32)
    for w in (32, 16, 8, 4, 2, 1):
        probe = plsc.load_gather(slots_v, [pos + (w - 1)])
        pos = jnp.where(probe < keys, pos + w, pos)
    return pos


def _sc_scan_kernel(src_hbm, dst_hbm, bitmap_hbm, slots_hbm, xsidx_hbm,
                    xtidx_hbm, x_hbm,
                    cnt_out, agg_out, pair_out, xs_out, xt_out,
                    esrc_v, edst_v, bitmap_v, slots_v, cnt_v, agg_v, pair_v,
                    rows_v, idx_v, xsidx_v, xtidx_v, xsrows_v, xtrows_v, sem):
    wid = lax.axis_index("s") * NC + lax.axis_index("c")
    base = wid * EPW

    li = lax.iota(jnp.int32, 16)
    zf = jnp.zeros((16,), jnp.float32)
    onehot = jnp.where(li == 0, 1.0, 0.0).astype(jnp.float32)

    def _z(n, ref):
        def body(i, _):
            ref[pl.ds(i * 16, 16)] = zf
            return 0
        lax.fori_loop(0, n // 16, body, 0)
    _z(64, cnt_v)
    _z(NSLOT * D + 16, agg_v)
    _z(NSLOT * NSLOT + 16, pair_v)

    pltpu.sync_copy(bitmap_hbm, bitmap_v)
    pltpu.sync_copy(slots_hbm, slots_v)
    pltpu.sync_copy(src_hbm.at[pl.ds(base, EPW)], esrc_v)
    pltpu.sync_copy(dst_hbm.at[pl.ds(base, EPW)], edst_v)

    def chunk(c, _):
        dstv = edst_v[pl.ds(c * 16, 16)]
        w = plsc.load_gather(bitmap_v, [lax.shift_right_logical(dstv, 5)])
        bit = lax.shift_right_logical(w, dstv & 31) & 1
        mi = jnp.where(bit != 0, 1, 0)
        anyhit = jnp.sum(mi)

        @pl.when(anyhit > 0)
        def _hit():
            srcv = esrc_v[pl.ds(c * 16, 16)]
            idx_v[...] = srcv
            cp = pltpu.async_copy(x_hbm.at[idx_v], rows_v, sem)
            pos = _binsearch64(slots_v, dstv)
            ws = plsc.load_gather(bitmap_v, [lax.shift_right_logical(srcv, 5)])
            bs = lax.shift_right_logical(ws, srcv & 31) & 1
            msl = jnp.where((bit != 0) & (bs != 0), 1, 0)
            upos = _binsearch64(slots_v, srcv)
            q = upos * NSLOT + pos
            cp.wait()
            for l in range(16):
                m_l = jnp.sum(jnp.where(li == l, mi, 0))

                @pl.when(m_l > 0)
                def _lane(l=l):
                    pos_l = jnp.sum(jnp.where(li == l, pos, 0))
                    a = pos_l * D
                    agg_v[pl.ds(a, 16)] = agg_v[pl.ds(a, 16)] + rows_v[l, pl.ds(0, 16)]
                    agg_v[pl.ds(a + 16, 16)] = (agg_v[pl.ds(a + 16, 16)]
                                                + rows_v[l, pl.ds(16, 16)])
                    cnt_v[pl.ds(pos_l, 16)] = cnt_v[pl.ds(pos_l, 16)] + onehot
                    s_l = jnp.sum(jnp.where(li == l, msl, 0))

                    @pl.when(s_l > 0)
                    def _pair():
                        q_l = jnp.sum(jnp.where(li == l, q, 0))
                        pair_v[pl.ds(q_l, 16)] = pair_v[pl.ds(q_l, 16)] + onehot
        return 0

    lax.fori_loop(0, NCHUNK, chunk, 0)

    pltpu.sync_copy(cnt_v.at[pl.ds(0, NSLOT)], cnt_out.at[wid])
    pltpu.sync_copy(agg_v.at[pl.ds(0, NSLOT * D)], agg_out.at[wid])
    pltpu.sync_copy(pair_v.at[pl.ds(0, NSLOT * NSLOT)], pair_out.at[wid])

    @pl.when(wid == 0)
    def _gather_rows():
        pltpu.sync_copy(xsidx_hbm, xsidx_v)
        pltpu.sync_copy(xtidx_hbm, xtidx_v)
        pltpu.async_copy(x_hbm.at[xsidx_v], xsrows_v, sem).wait()
        pltpu.async_copy(x_hbm.at[xtidx_v], xtrows_v, sem).wait()
        pltpu.sync_copy(xsrows_v, xs_out)
        pltpu.sync_copy(xtrows_v, xt_out)


def _sc_scan(src, dst, bitmap, slots64, xs_idx, xt_idx, x):
    mesh = plsc.VectorSubcoreMesh(core_axis_name="c", subcore_axis_name="s")
    f32 = jnp.float32
    call = functools.partial(
        pl.kernel,
        mesh=mesh,
        compiler_params=pltpu.CompilerParams(
            needs_layout_passes=False, use_tc_tiling_on_sc=False),
        out_type=[
            jax.ShapeDtypeStruct((NW, NSLOT), f32),
            jax.ShapeDtypeStruct((NW, NSLOT * D), f32),
            jax.ShapeDtypeStruct((NW, NSLOT * NSLOT), f32),
            jax.ShapeDtypeStruct((NSLOT, D), f32),
            jax.ShapeDtypeStruct((NTGT, D), f32),
        ],
        scratch_types=[
            pltpu.VMEM((EPW,), jnp.int32),
            pltpu.VMEM((EPW,), jnp.int32),
            pltpu.VMEM((BITW,), jnp.int32),
            pltpu.VMEM((SLOT64,), jnp.int32),
            pltpu.VMEM((64,), f32),
            pltpu.VMEM((NSLOT * D + 16,), f32),
            pltpu.VMEM((NSLOT * NSLOT + 16,), f32),
            pltpu.VMEM((16, D), f32),
            pltpu.VMEM((16,), jnp.int32),
            pltpu.VMEM((NSLOT,), jnp.int32),
            pltpu.VMEM((NTGT,), jnp.int32),
            pltpu.VMEM((NSLOT, D), f32),
            pltpu.VMEM((NTGT, D), f32),
            pltpu.SemaphoreType.DMA,
        ],
    )
    return call(_sc_scan_kernel)(src, dst, bitmap, slots64, xs_idx, xt_idx, x)


def _combine_kernel(cnt_ref, agg_ref, pair_ref, xs_ref, xt_ref,
                    memb_ref, tmatch_ref,
                    wl1, bl1, wr1, wl2, bl2, wr2, wl3, bl3, wr3,
                    fc1wt, fc1b, lng, lnb, pa, fc2w, fc2b, out_ref):
    f32 = jnp.float32
    cnt = jnp.sum(cnt_ref[...], axis=0)
    agg = jnp.sum(agg_ref[...], axis=0)
    pair = jnp.sum(pair_ref[...], axis=0)
    xs = xs_ref[...]
    memb1 = memb_ref[0, :][:, None]
    memb2 = memb_ref[1, :][:, None]
    memb3 = memb_ref[2, :][:, None]
    cntc = jnp.maximum(cnt, 1.0)[:, None]

    def dotT(a, b):
        return lax.dot_general(a, b, (((1,), (1,)), ((), ())),
                               preferred_element_type=f32)

    def dotTA(p, d):
        return lax.dot_general(p, d, (((0,), (0,)), ((), ())),
                               preferred_element_type=f32)

    h1 = jax.nn.relu(dotT(agg / cntc, wl1[...]) + bl1[...][None, :]
                     + dotT(xs, wr1[...]))
    d1 = jnp.where(memb1 > 0, h1 - xs, 0.0)
    agg2 = agg + dotTA(pair, d1)
    h2 = jax.nn.relu(dotT(agg2 / cntc, wl2[...]) + bl2[...][None, :]
                     + dotT(xs + d1, wr2[...]))
    d2 = jnp.where(memb2 > 0, h2 - xs, d1)
    agg3 = agg + dotTA(pair, d2)
    h3 = jax.nn.relu(dotT(agg3 / cntc, wl3[...]) + bl3[...][None, :]
                     + dotT(xs + d2, wr3[...]))
    d3 = jnp.where(memb3 > 0, h3 - xs, d2)

    x3t = xt_ref[...] + lax.dot_general(
        tmatch_ref[...], d3, (((1,), (0,)), ((), ())),
        preferred_element_type=f32)
    h = fc1b[...][None, :]
    for i in range(NTGT):
        h = h + lax.dot_general(x3t[i:i + 1, :], fc1wt[i],
                                (((1,), (0,)), ((), ())),
                                preferred_element_type=f32)
    mu = jnp.mean(h)
    var = jnp.mean((h - mu) ** 2)
    h = (h - mu) / jnp.sqrt(var + 1e-5) * lng[...][None, :] + lnb[...][None, :]
    h = jnp.where(h > 0, h, pa[0, 0] * h)
    h = dotT(h, fc2w[...]) + fc2b[...][None, :]
    out_ref[...] = jax.nn.softplus(h)


def _combine(cnt_p, agg_p, pair_p, xs, xt, memb, tmatch,
             Wl1, bl1, Wr1, Wl2, bl2, Wr2, Wl3, bl3, Wr3,
             fc1_Wt_pad, fc1_b, ln_g, ln_b, prelu_a, fc2_W, fc2_b):
    return pl.pallas_call(
        _combine_kernel,
        out_shape=jax.ShapeDtypeStruct((1, 5), jnp.float32),
    )(cnt_p, agg_p, pair_p, xs, xt, memb, tmatch,
      Wl1, bl1, Wr1, Wl2, bl2, Wr2, Wl3, bl3, Wr3,
      fc1_Wt_pad, fc1_b, ln_g, ln_b, prelu_a, fc2_W, fc2_b)


def kernel(x, edge_index, stage1_nodes, stage2_nodes, stage3_nodes,
           Wl1, bl1, Wr1, Wl2, bl2, Wr2, Wl3, bl3, Wr3,
           fc1_W, fc1_b, ln_g, ln_b, prelu_a, fc2_W, fc2_b):
    i32 = jnp.int32
    s_all = jnp.concatenate([stage1_nodes, stage2_nodes, stage3_nodes])
    n_all = s_all.shape[0]
    eq = s_all[:, None] == s_all[None, :]
    first = jnp.argmax(eq, axis=1)
    vals = jnp.where(first == jnp.arange(n_all), s_all, -1).astype(i32)
    slots64 = jnp.sort(jnp.concatenate(
        [vals, jnp.full((SLOT64 - n_all,), BIG, i32)]))
    slots = slots64[:NSLOT]
    valid = (slots >= 0) & (slots < BIG)

    word = jnp.where(valid, lax.shift_right_logical(slots, 5), 0)
    bit = jnp.where(valid, lax.shift_left(jnp.ones_like(slots), slots & 31), 0)
    bitmap = jnp.zeros((BITW,), i32).at[word].add(bit, mode="drop")

    memb = jnp.stack([
        ((slots[:, None] == stage1_nodes[None, :]).any(axis=1) & valid),
        ((slots[:, None] == stage2_nodes[None, :]).any(axis=1) & valid),
        ((slots[:, None] == stage3_nodes[None, :]).any(axis=1) & valid),
    ]).astype(jnp.float32)

    tgt = jnp.asarray(TARGET_IDS, i32)
    tgt_pad = jnp.concatenate([tgt, jnp.zeros((NTGT - tgt.shape[0],), i32)])
    tmatch = ((tgt[:, None] == slots[None, :]) & valid[None, :])
    tmatch = jnp.concatenate(
        [tmatch, jnp.zeros((NTGT - tgt.shape[0], NSLOT), bool)]
    ).astype(jnp.float32)

    xs_idx = jnp.where(valid, slots, 0)

    cnt_p, agg_p, pair_p, xs, xt = _sc_scan(
        edge_index[0], edge_index[1], bitmap, slots64, xs_idx, tgt_pad, x)
    agg_p = agg_p.reshape(NW, NSLOT, D)
    pair_p = pair_p.reshape(NW, NSLOT, NSLOT)

    fc1_Wt_pad = jnp.zeros((NTGT * D, 256), jnp.float32).at[:736, :].set(
        fc1_W.T).reshape(NTGT, D, 256)
    out = _combine(cnt_p, agg_p, pair_p, xs, xt, memb, tmatch,
                   Wl1, bl1, Wr1, Wl2, bl2, Wr2, Wl3, bl3, Wr3,
                   fc1_Wt_pad, fc1_b, ln_g, ln_b,
                   prelu_a.reshape(1, 1), fc2_W, fc2_b)
    return out.reshape(5)


# speedup vs baseline: 134.6842x; 1.3993x over previous
import functools

import jax
import jax.numpy as jnp
from jax import lax
from jax.experimental import pallas as pl
from jax.experimental.pallas import tpu as pltpu
from jax.experimental.pallas import tpu_sc as plsc

N = 100000
E = 1600000
D = 32
NSLOT = 48
SLOT64 = 64
NTGT = 24
BITW = 3136
BIG = 2 ** 30
GRP = 25

TARGET_IDS = (65, 66, 67, 73, 77, 78, 79, 80, 81, 82, 83,
              90, 91, 92, 93, 94, 95, 96, 97, 98, 99, 100, 101)

_info = plsc.get_sparse_core_info()
NC, NS = _info.num_cores, _info.num_subcores
NW = NC * NS
EPW = E // NW
NCHUNK = EPW // 16
NGRP = NCHUNK // GRP


def _binsearch64(slots_v, keys):
    pos = jnp.zeros((16,), jnp.int32)
    for w in (32, 16, 8, 4, 2, 1):
        probe = plsc.load_gather(slots_v, [pos + (w - 1)])
        pos = jnp.where(probe < keys, pos + w, pos)
    return pos


def _sc_scan_kernel(ei_hbm, slots_hbm, xsidx_hbm, xtidx_hbm, x_hbm,
                    cnt_out, agg_out, pair_out, xs_out, xt_out,
                    esrc_v, edst_v, bitmap_v, slots_v, mbuf_v,
                    cnt_v, agg_v, pair_v,
                    rows_v, idx_v, xsidx_v, xtidx_v, xsrows_v, xtrows_v, sem):
    wid = lax.axis_index("s") * NC + lax.axis_index("c")
    base = wid * EPW

    li = lax.iota(jnp.int32, 16)
    zf = jnp.zeros((16,), jnp.float32)
    zi = jnp.zeros((16,), jnp.int32)

    def _zf(i, _):
        cnt_v[pl.ds(i * 16, 16)] = zf
        return 0
    lax.fori_loop(0, 4, _zf, 0)

    def _za(i, _):
        agg_v[i, pl.ds(0, 16)] = zf
        agg_v[i, pl.ds(16, 16)] = zf
        pair_v[i, pl.ds(0, 16)] = zf
        pair_v[i, pl.ds(16, 16)] = zf
        pair_v[i, pl.ds(32, 16)] = zf
        return 0
    lax.fori_loop(0, NSLOT, _za, 0)

    def _zb(i, _):
        bitmap_v[pl.ds(i * 16, 16)] = zi
        return 0
    lax.fori_loop(0, BITW // 16, _zb, 0)

    pltpu.sync_copy(slots_hbm, slots_v)
    pltpu.sync_copy(ei_hbm.at[0, pl.ds(base, EPW)], esrc_v)
    pltpu.sync_copy(ei_hbm.at[1, pl.ds(base, EPW)], edst_v)

    for g in range(3):
        sv = slots_v[pl.ds(g * 16, 16)]
        val = (sv >= 0) & (sv < BIG)
        wv = jnp.where(val, lax.shift_right_logical(sv, 5), 0)
        bv = jnp.where(val, lax.shift_left(jnp.ones_like(sv), sv & 31), 0)
        for l in range(16):
            w_l = jnp.sum(jnp.where(li == l, wv, 0))
            b_l = jnp.sum(jnp.where(li == l, bv, 0))
            w_al = w_l & ~15
            win = bitmap_v[pl.ds(w_al, 16)]
            bitmap_v[pl.ds(w_al, 16)] = win | jnp.where(li == (w_l & 15), b_l, 0)

    def _hit_chunk(c):
        dstv = edst_v[pl.ds(c * 16, 16)]
        srcv = esrc_v[pl.ds(c * 16, 16)]
        w = plsc.load_gather(bitmap_v, [lax.shift_right_logical(dstv, 5)])
        mi = lax.shift_right_logical(w, dstv & 31) & 1
        idx_v[...] = srcv
        cp = pltpu.async_copy(x_hbm.at[idx_v], rows_v, sem)
        pos = _binsearch64(slots_v, dstv)
        ws = plsc.load_gather(bitmap_v, [lax.shift_right_logical(srcv, 5)])
        bs = lax.shift_right_logical(ws, srcv & 31) & 1
        msl = mi & bs
        upos = _binsearch64(slots_v, srcv)
        cp.wait()
        for l in range(16):
            m_l = jnp.sum(jnp.where(li == l, mi, 0))

            @pl.when(m_l > 0)
            def _lane(l=l):
                pos_l = jnp.sum(jnp.where(li == l, pos, 0))
                agg_v[pos_l, pl.ds(0, 16)] = (
                    agg_v[pos_l, pl.ds(0, 16)] + rows_v[l, pl.ds(0, 16)])
                agg_v[pos_l, pl.ds(16, 16)] = (
                    agg_v[pos_l, pl.ds(16, 16)] + rows_v[l, pl.ds(16, 16)])
                p_al = pos_l & ~15
                cnt_v[pl.ds(p_al, 16)] = (
                    cnt_v[pl.ds(p_al, 16)]
                    + jnp.where(li == (pos_l & 15), 1.0, 0.0))
                s_l = jnp.sum(jnp.where(li == l, msl, 0))

                @pl.when(s_l > 0)
                def _pair():
                    u_l = jnp.sum(jnp.where(li == l, upos, 0))
                    pair_v[u_l, pl.ds(p_al, 16)] = (
                        pair_v[u_l, pl.ds(p_al, 16)]
                        + jnp.where(li == (pos_l & 15), 1.0, 0.0))

    def group(g, _):
        c0 = g * GRP
        acc = zi
        for j in range(GRP):
            dstv = edst_v[pl.ds((c0 + j) * 16, 16)]
            w = plsc.load_gather(bitmap_v, [lax.shift_right_logical(dstv, 5)])
            mi = lax.shift_right_logical(w, dstv & 31) & 1
            mbuf_v[pl.ds(j * 16, 16)] = mi
            acc = acc | mi
        anyg = jnp.sum(acc)

        @pl.when(anyg > 0)
        def _rescan():
            def rchunk(j, _):
                mi = mbuf_v[pl.ds(j * 16, 16)]
                nh = jnp.sum(mi)

                @pl.when(nh > 0)
                def _h():
                    _hit_chunk(c0 + j)
                return 0
            lax.fori_loop(0, GRP, rchunk, 0)
        return 0

    lax.fori_loop(0, NGRP, group, 0)

    pltpu.sync_copy(cnt_v.at[pl.ds(0, NSLOT)], cnt_out.at[wid])
    pltpu.sync_copy(agg_v, agg_out.at[pl.ds(wid * NSLOT, NSLOT)])
    pltpu.sync_copy(pair_v, pair_out.at[pl.ds(wid * NSLOT, NSLOT)])

    @pl.when(wid == 0)
    def _gather_rows():
        pltpu.sync_copy(xsidx_hbm, xsidx_v)
        pltpu.sync_copy(xtidx_hbm, xtidx_v)
        pltpu.async_copy(x_hbm.at[xsidx_v], xsrows_v, sem).wait()
        pltpu.async_copy(x_hbm.at[xtidx_v], xtrows_v, sem).wait()
        pltpu.sync_copy(xsrows_v, xs_out)
        pltpu.sync_copy(xtrows_v, xt_out)


def _sc_scan(edge_index, slots64, xs_idx, xt_idx, x):
    mesh = plsc.VectorSubcoreMesh(core_axis_name="c", subcore_axis_name="s")
    f32 = jnp.float32
    call = functools.partial(
        pl.kernel,
        mesh=mesh,
        compiler_params=pltpu.CompilerParams(
            needs_layout_passes=False, use_tc_tiling_on_sc=False),
        out_type=[
            jax.ShapeDtypeStruct((NW, NSLOT), f32),
            jax.ShapeDtypeStruct((NW * NSLOT, D), f32),
            jax.ShapeDtypeStruct((NW * NSLOT, NSLOT), f32),
            jax.ShapeDtypeStruct((NSLOT, D), f32),
            jax.ShapeDtypeStruct((NTGT, D), f32),
        ],
        scratch_types=[
            pltpu.VMEM((EPW,), jnp.int32),
            pltpu.VMEM((EPW,), jnp.int32),
            pltpu.VMEM((BITW,), jnp.int32),
            pltpu.VMEM((SLOT64,), jnp.int32),
            pltpu.VMEM((GRP * 16,), jnp.int32),
            pltpu.VMEM((64,), f32),
            pltpu.VMEM((NSLOT, D), f32),
            pltpu.VMEM((NSLOT, NSLOT), f32),
            pltpu.VMEM((16, D), f32),
            pltpu.VMEM((16,), jnp.int32),
            pltpu.VMEM((NSLOT,), jnp.int32),
            pltpu.VMEM((NTGT,), jnp.int32),
            pltpu.VMEM((NSLOT, D), f32),
            pltpu.VMEM((NTGT, D), f32),
            pltpu.SemaphoreType.DMA,
        ],
    )
    return call(_sc_scan_kernel)(edge_index, slots64, xs_idx, xt_idx, x)


def _combine_kernel(cnt_ref, agg_ref, pair_ref, xs_ref, xt_ref,
                    memb_ref, tmatch_ref,
                    wl1, bl1, wr1, wl2, bl2, wr2, wl3, bl3, wr3,
                    fc1wt, fc1b, lng, lnb, pa, fc2w, fc2b, out_ref):
    f32 = jnp.float32
    col = lax.broadcasted_iota(jnp.int32, (NSLOT, NW * NSLOT), 1)
    row = lax.broadcasted_iota(jnp.int32, (NSLOT, NW * NSLOT), 0)
    rm = jnp.where(col % NSLOT == row, 1.0, 0.0).astype(f32)

    def mm(a, b):
        return lax.dot_general(a, b, (((1,), (0,)), ((), ())),
                               preferred_element_type=f32)

    def dotT(a, b):
        return lax.dot_general(a, b, (((1,), (1,)), ((), ())),
                               preferred_element_type=f32)

    def dotTA(p, d):
        return lax.dot_general(p, d, (((0,), (0,)), ((), ())),
                               preferred_element_type=f32)

    cnt = jnp.sum(cnt_ref[...], axis=0)
    agg = mm(rm, agg_ref[...])
    pair = mm(rm, pair_ref[...])
    xs = xs_ref[...]
    memb1 = memb_ref[0, :][:, None]
    memb2 = memb_ref[1, :][:, None]
    memb3 = memb_ref[2, :][:, None]
    cntc = jnp.maximum(cnt, 1.0)[:, None]

    h1 = jax.nn.relu(dotT(agg / cntc, wl1[...]) + bl1[...][None, :]
                     + dotT(xs, wr1[...]))
    d1 = jnp.where(memb1 > 0, h1 - xs, 0.0)
    agg2 = agg + dotTA(pair, d1)
    h2 = jax.nn.relu(dotT(agg2 / cntc, wl2[...]) + bl2[...][None, :]
                     + dotT(xs + d1, wr2[...]))
    d2 = jnp.where(memb2 > 0, h2 - xs, d1)
    agg3 = agg + dotTA(pair, d2)
    h3 = jax.nn.relu(dotT(agg3 / cntc, wl3[...]) + bl3[...][None, :]
                     + dotT(xs + d2, wr3[...]))
    d3 = jnp.where(memb3 > 0, h3 - xs, d2)

    x3t = xt_ref[...] + lax.dot_general(
        tmatch_ref[...], d3, (((1,), (0,)), ((), ())),
        preferred_element_type=f32)
    h = fc1b[...][None, :]
    for i in range(NTGT):
        h = h + lax.dot_general(x3t[i:i + 1, :], fc1wt[i],
                                (((1,), (0,)), ((), ())),
                                preferred_element_type=f32)
    mu = jnp.mean(h)
    var = jnp.mean((h - mu) ** 2)
    h = (h - mu) / jnp.sqrt(var + 1e-5) * lng[...][None, :] + lnb[...][None, :]
    h = jnp.where(h > 0, h, pa[0, 0] * h)
    h = dotT(h, fc2w[...]) + fc2b[...][None, :]
    out_ref[...] = jax.nn.softplus(h)


def _combine(cnt_p, agg_p, pair_p, xs, xt, memb, tmatch,
             Wl1, bl1, Wr1, Wl2, bl2, Wr2, Wl3, bl3, Wr3,
             fc1_Wt_pad, fc1_b, ln_g, ln_b, prelu_a, fc2_W, fc2_b):
    return pl.pallas_call(
        _combine_kernel,
        out_shape=jax.ShapeDtypeStruct((1, 5), jnp.float32),
    )(cnt_p, agg_p, pair_p, xs, xt, memb, tmatch,
      Wl1, bl1, Wr1, Wl2, bl2, Wr2, Wl3, bl3, Wr3,
      fc1_Wt_pad, fc1_b, ln_g, ln_b, prelu_a, fc2_W, fc2_b)


def kernel(x, edge_index, stage1_nodes, stage2_nodes, stage3_nodes,
           Wl1, bl1, Wr1, Wl2, bl2, Wr2, Wl3, bl3, Wr3,
           fc1_W, fc1_b, ln_g, ln_b, prelu_a, fc2_W, fc2_b):
    i32 = jnp.int32
    s_all = jnp.concatenate([stage1_nodes, stage2_nodes, stage3_nodes])
    n_all = s_all.shape[0]
    eq = s_all[:, None] == s_all[None, :]
    first = jnp.argmax(eq, axis=1)
    vals = jnp.where(first == jnp.arange(n_all), s_all, -1).astype(i32)
    slots64 = jnp.sort(jnp.concatenate(
        [vals, jnp.full((SLOT64 - n_all,), BIG, i32)]))
    slots = slots64[:NSLOT]
    valid = (slots >= 0) & (slots < BIG)

    memb = jnp.stack([
        ((slots[:, None] == stage1_nodes[None, :]).any(axis=1) & valid),
        ((slots[:, None] == stage2_nodes[None, :]).any(axis=1) & valid),
        ((slots[:, None] == stage3_nodes[None, :]).any(axis=1) & valid),
    ]).astype(jnp.float32)

    tgt = jnp.asarray(TARGET_IDS, i32)
    tgt_pad = jnp.concatenate([tgt, jnp.zeros((NTGT - tgt.shape[0],), i32)])
    tmatch = ((tgt[:, None] == slots[None, :]) & valid[None, :])
    tmatch = jnp.concatenate(
        [tmatch, jnp.zeros((NTGT - tgt.shape[0], NSLOT), bool)]
    ).astype(jnp.float32)

    xs_idx = jnp.where(valid, slots, 0)

    cnt_p, agg_p, pair_p, xs, xt = _sc_scan(
        edge_index, slots64, xs_idx, tgt_pad, x)

    fc1_Wt_pad = jnp.zeros((NTGT * D, 256), jnp.float32).at[:736, :].set(
        fc1_W.T).reshape(NTGT, D, 256)
    out = _combine(cnt_p, agg_p, pair_p, xs, xt, memb, tmatch,
                   Wl1, bl1, Wr1, Wl2, bl2, Wr2, Wl3, bl3, Wr3,
                   fc1_Wt_pad, fc1_b, ln_g, ln_b,
                   prelu_a.reshape(1, 1), fc2_W, fc2_b)
    return out.reshape(5)
